# baseline (speedup 1.0000x reference)
import jax
import jax.numpy as jnp
from jax import lax
from jax.experimental import pallas as pl
from jax.experimental.pallas import tpu as pltpu
from jax.experimental.pallas import tpu_sc as plsc

NC = 2
NS = 16
NW = NC * NS
L = 16

BINS = 256
ROWS = 16 * 512
ROW_W = 512
R_PER_W = ROWS // NW
CH_R = 16
NCHUNK = R_PER_W // CH_R
TABLE = BINS * L


def _sc_hist(img_hbm, out_hbm, buf0, buf1, table, hist, sem0, sem1):
    c = lax.axis_index("c")
    s = lax.axis_index("s")
    wid = s * NC + c
    row0 = wid * R_PER_W

    zeros = jnp.zeros((L,), jnp.float32)
    ones = jnp.ones((L,), jnp.float32)
    magic = (lax.iota(jnp.int32, L) * BINS).astype(jnp.float32) + 8388608.0
    expbits = jnp.int32(0x4B000000)

    bufs = (buf0, buf1)
    sems = (sem0, sem1)

    def cp(ci):
        return pltpu.make_async_copy(
            img_hbm.at[pl.ds(row0 + ci * CH_R, CH_R), :], bufs[ci % 2], sems[ci % 2]
        )

    cp(0).start()
    cp(1).start()

    @plsc.parallel_loop(0, TABLE // L, unroll=4)
    def _zero(i):
        table[pl.ds(i * L, L)] = zeros

    for ci in range(NCHUNK):
        cp(ci).wait()
        cur = bufs[ci % 2]

        @plsc.parallel_loop(0, CH_R * (ROW_W // L), unroll=8)
        def _vec(i, cur=cur):
            r = lax.shift_right_logical(i, 5)
            off = lax.shift_left(i & 31, 4)
            v = cur[r, pl.ds(off, L)]
            idx = plsc.bitcast(v + magic, jnp.int32) - expbits
            plsc.addupdate_scatter(table, (idx,), ones)

        if ci + 2 < NCHUNK:
            cp(ci + 2).start()

    @plsc.parallel_loop(0, BINS // L, unroll=2)
    def _red(j):
        vals = [table[pl.ds(lane * BINS + j * L, L)] for lane in range(L)]
        while len(vals) > 1:
            vals = [a + b for a, b in zip(vals[::2], vals[1::2])]
        hist[pl.ds(j * L, L)] = vals[0]

    pltpu.sync_copy(hist, out_hbm.at[wid])


_sc_call = pl.kernel(
    _sc_hist,
    out_type=jax.ShapeDtypeStruct((NW, BINS), jnp.float32),
    mesh=plsc.VectorSubcoreMesh(
        core_axis_name="c", subcore_axis_name="s", num_cores=NC, num_subcores=NS
    ),
    scratch_types=[
        pltpu.VMEM((CH_R, ROW_W), jnp.float32),
        pltpu.VMEM((CH_R, ROW_W), jnp.float32),
        pltpu.VMEM((TABLE,), jnp.float32),
        pltpu.VMEM((BINS,), jnp.float32),
        pltpu.SemaphoreType.DMA,
        pltpu.SemaphoreType.DMA,
    ],
    compiler_params=pltpu.CompilerParams(needs_layout_passes=False),
)


def _make_tc_entropy(batch, temp):
    inv_temp = 1.0 / float(temp)
    log2e = 1.4426950408889634

    def _tc_entropy(parts_ref, out_ref):
        h = jnp.sum(parts_ref[...], axis=0)
        tmp = h * inv_temp
        safe = jnp.where(tmp > 0, tmp, 1.0)
        contrib = jnp.where(tmp > 0, tmp * (jnp.log(safe) * log2e), 0.0)
        res = -jnp.sum(contrib)
        out_ref[...] = jnp.reshape(float(batch) * (8.0 - res), (1, 1))

    return pl.pallas_call(
        _tc_entropy,
        out_shape=jax.ShapeDtypeStruct((1, 1), jnp.float32),
    )


def kernel(img):
    batch, height, width = img.shape
    rows = img.reshape(batch * height, width)
    parts = _sc_call(rows)
    out = _make_tc_entropy(batch, height * width)(parts)
    return out.reshape(())


# speedup vs baseline: 1.0082x; 1.0082x over previous
import jax
import jax.numpy as jnp
from jax import lax
from jax.experimental import pallas as pl
from jax.experimental.pallas import tpu as pltpu
from jax.experimental.pallas import tpu_sc as plsc

NC = 2
NS = 16
NW = NC * NS
L = 16

BINS = 256
ROWS = 16 * 512
ROW_W = 512
R_PER_W = ROWS // NW
CH_R = 32
NCHUNK = R_PER_W // CH_R
TABLE = BINS * L


def _sc_hist(img_hbm, out_hbm, buf0, buf1, table, hist, sem0, sem1):
    c = lax.axis_index("c")
    s = lax.axis_index("s")
    wid = s * NC + c
    row0 = wid * R_PER_W

    zeros = jnp.zeros((L,), jnp.float32)
    ones = jnp.ones((L,), jnp.float32)
    magic = (lax.iota(jnp.int32, L) * BINS).astype(jnp.float32) + 8388608.0
    expbits = jnp.int32(0x4B000000)

    bufs = (buf0, buf1)
    sems = (sem0, sem1)

    def cp(ci):
        return pltpu.make_async_copy(
            img_hbm.at[pl.ds(row0 + ci * CH_R, CH_R), :], bufs[ci % 2], sems[ci % 2]
        )

    cp(0).start()
    cp(1).start()

    @plsc.parallel_loop(0, TABLE // L, unroll=4)
    def _zero(i):
        table[pl.ds(i * L, L)] = zeros

    for ci in range(NCHUNK):
        cp(ci).wait()
        cur = bufs[ci % 2]

        @plsc.parallel_loop(0, CH_R * (ROW_W // L), unroll=8)
        def _vec(i, cur=cur):
            r = lax.shift_right_logical(i, 5)
            off = lax.shift_left(i & 31, 4)
            v = cur[r, pl.ds(off, L)]
            idx = plsc.bitcast(v + magic, jnp.int32) - expbits
            plsc.addupdate_scatter(table, (idx,), ones)

        if ci + 2 < NCHUNK:
            cp(ci + 2).start()

    @plsc.parallel_loop(0, BINS // L, unroll=2)
    def _red(j):
        vals = [table[pl.ds(lane * BINS + j * L, L)] for lane in range(L)]
        while len(vals) > 1:
            vals = [a + b for a, b in zip(vals[::2], vals[1::2])]
        hist[pl.ds(j * L, L)] = vals[0]

    pltpu.sync_copy(hist, out_hbm.at[wid])


_sc_call = pl.kernel(
    _sc_hist,
    out_type=jax.ShapeDtypeStruct((NW, BINS), jnp.float32),
    mesh=plsc.VectorSubcoreMesh(
        core_axis_name="c", subcore_axis_name="s", num_cores=NC, num_subcores=NS
    ),
    scratch_types=[
        pltpu.VMEM((CH_R, ROW_W), jnp.float32),
        pltpu.VMEM((CH_R, ROW_W), jnp.float32),
        pltpu.VMEM((TABLE,), jnp.float32),
        pltpu.VMEM((BINS,), jnp.float32),
        pltpu.SemaphoreType.DMA,
        pltpu.SemaphoreType.DMA,
    ],
    compiler_params=pltpu.CompilerParams(needs_layout_passes=False),
)


def _make_tc_entropy(batch, temp):
    inv_temp = 1.0 / float(temp)
    log2e = 1.4426950408889634

    def _tc_entropy(parts_ref, out_ref):
        h = jnp.sum(parts_ref[...], axis=0)
        tmp = h * inv_temp
        safe = jnp.where(tmp > 0, tmp, 1.0)
        contrib = jnp.where(tmp > 0, tmp * (jnp.log(safe) * log2e), 0.0)
        res = -jnp.sum(contrib)
        out_ref[...] = jnp.reshape(float(batch) * (8.0 - res), (1, 1))

    return pl.pallas_call(
        _tc_entropy,
        out_shape=jax.ShapeDtypeStruct((1, 1), jnp.float32),
    )


def kernel(img):
    batch, height, width = img.shape
    rows = img.reshape(batch * height, width)
    parts = _sc_call(rows)
    out = _make_tc_entropy(batch, height * width)(parts)
    return out.reshape(())
